# baseline (speedup 1.0000x reference)
import functools

import jax
import jax.numpy as jnp
from jax import lax
from jax.experimental import pallas as pl
from jax.experimental.pallas import tpu as pltpu
from jax.experimental.pallas import tpu_sc as plsc

PACKET = 4
LANES = 16
NUM_CORES = 2
NUM_SUBCORES = 16
NW = NUM_CORES * NUM_SUBCORES
NCHUNK = 4


def kernel(b1, b2, b3, indices):
    B = b1.shape[0]
    S = b1.shape[1]
    C = b1.shape[2]
    F = S * C
    N = 3 * F
    CHUNK = N // NCHUNK
    CT = C // 128

    def view5(x):
        return (x.transpose(1, 0, 2)
                 .reshape(S, B // 8, 8, CT, 128)
                 .transpose(0, 1, 3, 2, 4))

    v1, v2, v3 = view5(b1), view5(b2), view5(b3)
    idx3 = lax.bitcast_convert_type(indices, jnp.float32).reshape(
        3 * S, CT, 128)

    rows_per_w = B // NW
    pairs = rows_per_w // 2

    mesh = plsc.VectorSubcoreMesh(core_axis_name="c", subcore_axis_name="s")

    @functools.partial(
        pl.kernel,
        mesh=mesh,
        out_type=jax.ShapeDtypeStruct((B * N // 128, 128), jnp.float32),
        scratch_types=[
            pltpu.VMEM((N,), jnp.int32),
            pltpu.VMEM((3 * S, CT, 128), jnp.float32),
            pltpu.VMEM((3 * S, CT, 128), jnp.float32),
            pltpu.VMEM((CHUNK // 128, 128), jnp.float32),
            pltpu.VMEM((CHUNK // 128, 128), jnp.float32),
            pltpu.SemaphoreType.DMA,
            pltpu.SemaphoreType.DMA,
            pltpu.SemaphoreType.DMA,
            pltpu.SemaphoreType.DMA,
        ],
        compiler_params=pltpu.CompilerParams(needs_layout_passes=False),
    )
    def interleave(v1_hbm, v2_hbm, v3_hbm, idx_hbm, out_hbm,
                   idx_v, in0, in1, ob0, ob1, is0, is1, os0, os1):
        wid = lax.axis_index("s") * NUM_CORES + lax.axis_index("c")
        row0 = wid * rows_per_w
        lanes = lax.iota(jnp.int32, LANES)

        pltpu.sync_copy(idx_hbm, in0)

        @plsc.parallel_loop(0, N // LANES, 1, unroll=8)
        def _xform(kv):
            tj = kv >> 5
            rem = kv & 31
            p = rem >> 3
            jbase = (rem & 7) * LANES
            n = tj * 512 + 4 * (jbase + lanes) + p
            raw = plsc.load_gather(
                in0, [n >> 12, (n >> 7) & (CT - 1), n & 127])
            idx_v[pl.ds(kv * LANES, LANES)] = plsc.bitcast(raw, jnp.int32)

        def start_in(r, inbuf, sem):
            rt = r >> 3
            ri = r & 7
            pltpu.async_copy(v1_hbm.at[:, rt, :, ri, :],
                             inbuf.at[pl.ds(0, S)], sem)
            pltpu.async_copy(v2_hbm.at[:, rt, :, ri, :],
                             inbuf.at[pl.ds(S, S)], sem)
            pltpu.async_copy(v3_hbm.at[:, rt, :, ri, :],
                             inbuf.at[pl.ds(2 * S, S)], sem)

        def wait_in(r, inbuf, sem):
            rt = r >> 3
            ri = r & 7
            pltpu.make_async_copy(v1_hbm.at[:, rt, :, ri, :],
                                  inbuf.at[pl.ds(0, S)], sem).wait()
            pltpu.make_async_copy(v2_hbm.at[:, rt, :, ri, :],
                                  inbuf.at[pl.ds(S, S)], sem).wait()
            pltpu.make_async_copy(v3_hbm.at[:, rt, :, ri, :],
                                  inbuf.at[pl.ds(2 * S, S)], sem).wait()

        def gather_chunk(inbuf, outbuf, c):
            @plsc.parallel_loop(0, CHUNK // LANES, 1, unroll=8)
            def _gather(j):
                tidx = idx_v[pl.ds(c * CHUNK + j * LANES, LANES)]
                outbuf[j >> 3, pl.ds((j & 7) * LANES, LANES)] = (
                    plsc.load_gather(
                        inbuf,
                        [tidx >> 12, (tidx >> 7) & (CT - 1), tidx & 127]))

        def out_slice(r, c):
            return out_hbm.at[
                pl.ds(r * (N // 128) + c * (CHUNK // 128), CHUNK // 128), :]

        def drain_out(r, c, outbuf, sem):
            pltpu.make_async_copy(outbuf, out_slice(r, c), sem).wait()

        start_in(row0, in0, is0)
        start_in(row0 + 1, in1, is1)

        def pair_body(p, carry):
            for half, (inbuf, isem) in enumerate(((in0, is0), (in1, is1))):
                r = row0 + 2 * p + half
                wait_in(r, inbuf, isem)
                for c in range(NCHUNK):
                    outbuf, osem = (ob0, os0) if c % 2 == 0 else (ob1, os1)
                    g = half * NCHUNK + c
                    if g >= 2:
                        pc = (g - 2) % NCHUNK
                        prev_row = row0 + 2 * p + (g - 2) // NCHUNK
                        drain_out(prev_row, pc, outbuf, osem)
                    else:
                        @pl.when(p > 0)
                        def _():
                            drain_out(row0 + 2 * p - 1, c + 2, outbuf, osem)
                    gather_chunk(inbuf, outbuf, c)
                    pltpu.async_copy(outbuf, out_slice(r, c), osem)
                nxt = r + 2
                @pl.when(nxt < row0 + rows_per_w)
                def _():
                    start_in(nxt, inbuf, isem)
            return carry

        lax.fori_loop(0, pairs, pair_body, 0)

        last = row0 + rows_per_w - 1
        drain_out(last, 2, ob0, os0)
        drain_out(last, 3, ob1, os1)

    out = interleave(v1, v2, v3, idx3)
    return (out.reshape(B, N // (PACKET * 128), PACKET, 128)
               .transpose(0, 1, 3, 2)
               .reshape(B, N // PACKET, PACKET))
